# Initial kernel scaffold; baseline (speedup 1.0000x reference)
import jax
import jax.numpy as jnp
from jax.experimental import pallas as pl


def kernel(x, W_reg, b_reg, W_conf, b_conf):
    raise NotImplementedError("write your pallas kernel here")


# speedup vs baseline: 1.1121x; 1.1121x over previous
import functools

import jax
import jax.numpy as jnp
from jax.experimental import pallas as pl
from jax.experimental.pallas import tpu as pltpu

_K = 100


def _maps_kernel(w_ref, b_ref, x_ref, out_ref):
    y = jnp.dot(w_ref[...], x_ref[0], preferred_element_type=jnp.float32)
    y = y + b_ref[...]
    reg0 = y[0]
    reg1 = y[1]
    conf = jax.nn.sigmoid(y[2])
    score = jnp.sqrt(reg0 * reg0 + reg1 * reg1) * conf
    out_ref[0] = jnp.stack([reg0, reg1, conf, score], axis=0)


def _compute_maps(x, W_reg, b_reg, W_conf, b_conf, tn=9216):
    B, C, H, W = x.shape
    N = H * W
    xf = x.reshape(B, C, N)
    w = jnp.zeros((8, C), jnp.float32).at[0:2].set(W_reg).at[2].set(W_conf[0])
    b = jnp.zeros((8, 1), jnp.float32).at[0:2, 0].set(b_reg).at[2, 0].set(b_conf[0])
    grid = (B, N // tn)
    return pl.pallas_call(
        _maps_kernel,
        grid=grid,
        in_specs=[
            pl.BlockSpec((8, C), lambda b_, i: (0, 0)),
            pl.BlockSpec((8, 1), lambda b_, i: (0, 0)),
            pl.BlockSpec((1, C, tn), lambda b_, i: (b_, 0, i)),
        ],
        out_specs=pl.BlockSpec((1, 4, tn), lambda b_, i: (b_, 0, i)),
        out_shape=jax.ShapeDtypeStruct((B, 4, N), jnp.float32),
        compiler_params=pltpu.CompilerParams(
            dimension_semantics=("parallel", "arbitrary"),
        ),
    )(w, b, xf)


def kernel(x, W_reg, b_reg, W_conf, b_conf):
    B, C, H, W = x.shape
    N = H * W
    maps = _compute_maps(x, W_reg, b_reg, W_conf, b_conf)
    scores = maps[:, 3, :]
    _, topk_idx = jax.lax.top_k(scores, _K)
    sel = jnp.take_along_axis(maps[:, 0:3, :], topk_idx[:, None, :], axis=2)
    gx = (topk_idx // W).astype(jnp.float32)
    gy = (topk_idx % W).astype(jnp.float32)
    out = jnp.stack([gx + sel[:, 0], gy + sel[:, 1], sel[:, 2]], axis=-1)
    return out


# speedup vs baseline: 1.2653x; 1.1377x over previous
import functools

import jax
import jax.numpy as jnp
from jax import lax
from jax.experimental import pallas as pl
from jax.experimental.pallas import tpu as pltpu
from jax.experimental.pallas import tpu_sc as plsc

_K = 100
_B = 4
_HW = 384
_N = _HW * _HW
_ROWS = _N // 128
_TROWS = _ROWS // 16
_NVPT = _TROWS * 8
_HB = 8192
_SHIFT = 19
_CCAP = 512
_PCAP = 128
_MCAP = 2080
_SELP = 112
_OPAD = 512
_BIG = 2**30


def _maps_kernel(w_ref, b_ref, x_ref, r0_ref, r1_ref, cf_ref, sc_ref):
    y = jnp.dot(w_ref[...], x_ref[0], preferred_element_type=jnp.float32)
    y = y + b_ref[...]
    reg0 = y[0]
    reg1 = y[1]
    conf = jax.nn.sigmoid(y[2])
    score = jnp.sqrt(reg0 * reg0 + reg1 * reg1) * conf
    tr = r0_ref.shape[0]
    r0_ref[...] = reg0.reshape(tr, 128)
    r1_ref[...] = reg1.reshape(tr, 128)
    cf_ref[...] = conf.reshape(tr, 128)
    sc_ref[...] = score.reshape(tr, 128)


def _compute_maps(x, W_reg, b_reg, W_conf, b_conf, tn=9216):
    B, C, H, W = x.shape
    N = H * W
    xf = x.reshape(B, C, N)
    w = jnp.zeros((8, C), jnp.float32).at[0:2].set(W_reg).at[2].set(W_conf[0])
    b = jnp.zeros((8, 1), jnp.float32).at[0:2, 0].set(b_reg).at[2, 0].set(b_conf[0])
    g = N // tn
    tr = tn // 128
    plane = jax.ShapeDtypeStruct((B * N // 128, 128), jnp.float32)
    out_spec = pl.BlockSpec((tr, 128), lambda b_, i: (b_ * g + i, 0))
    return pl.pallas_call(
        _maps_kernel,
        grid=(B, g),
        in_specs=[
            pl.BlockSpec((8, C), lambda b_, i: (0, 0)),
            pl.BlockSpec((8, 1), lambda b_, i: (0, 0)),
            pl.BlockSpec((1, C, tn), lambda b_, i: (b_, 0, i)),
        ],
        out_specs=[out_spec, out_spec, out_spec, out_spec],
        out_shape=[plane, plane, plane, plane],
        compiler_params=pltpu.CompilerParams(
            dimension_semantics=("parallel", "arbitrary"),
        ),
    )(w, b, xf)


def _sel_body(r0_hbm, r1_hbm, cf_hbm, sc_hbm, out_hbm,
              chunk, hist, zidx, cands_s, cands_i, cntv,
              mbuf_s, mbuf_i, mcnt, merged_s, merged_i,
              sel_n, sel_r, g0, g1, g2, outbuf,
              sh_hist, sh_cs, sh_ci, sh_cnt, sem):
    cid = lax.axis_index("c")
    sid = lax.axis_index("s")
    iota = lax.iota(jnp.int32, 16)
    zeros16i = jnp.zeros((16,), jnp.int32)
    ones16i = jnp.ones((16,), jnp.int32)

    zidx[...] = zeros16i

    for t in range(2):
        b = cid * 2 + t

        def zh(i, c):
            hist[0, pl.ds(i * 16, 16)] = zeros16i
            return c
        lax.fori_loop(0, _HB // 16, zh, 0)

        @pl.when(sid == 0)
        def _():
            pltpu.sync_copy(hist, sh_hist)

        row0 = b * _ROWS + sid * _TROWS
        pltpu.sync_copy(sc_hbm.at[pl.ds(row0, _TROWS), :], chunk)
        plsc.subcore_barrier()

        def hacc(i, c):
            r = i // 8
            c4 = (i % 8) * 16
            v = chunk[r, pl.ds(c4, 16)]
            bits = plsc.bitcast(v, jnp.int32)
            bkt = lax.shift_right_logical(bits, _SHIFT)
            plsc.addupdate_scatter(hist, [zeros16i, bkt], ones16i)
            return c
        lax.fori_loop(0, _NVPT, hacc, 0)

        pltpu.sync_copy(hist, sh_hist.at[zidx.at[pl.ds(0, 1)]], add=True)
        plsc.subcore_barrier()

        pltpu.sync_copy(sh_hist, hist)

        def scan_body(i, carry):
            acc, bstar = carry
            j = _HB // 16 - 1 - i
            v = hist[0, pl.ds(j * 16, 16)]
            rv = lax.rev(v, (0,))
            cs = jnp.cumsum(rv)
            suf = lax.rev(cs, (0,)) + acc
            bidx = j * 16 + iota
            cand = jnp.where(suf >= _K, bidx, -1)
            bstar = jnp.maximum(bstar, jnp.max(cand))
            return (acc + cs[15], bstar)

        _, bstar = lax.fori_loop(0, _HB // 16, scan_body,
                                 (jnp.int32(0), jnp.int32(-1)))
        thresh_bits = bstar << _SHIFT

        base_n = sid * (_N // 16)

        def comp(i, off):
            r = i // 8
            c4 = (i % 8) * 16
            v = chunk[r, pl.ds(c4, 16)]
            bits = plsc.bitcast(v, jnp.int32)
            msk = bits >= thresh_bits
            plsc.store_compressed(cands_s.at[pl.ds(off, 16)], v, mask=msk)
            gidx = base_n + i * 16 + iota
            plsc.store_compressed(cands_i.at[pl.ds(off, 16)], gidx, mask=msk)
            pc = plsc.all_reduce_population_count(msk)
            return jnp.minimum(off + pc[0], _CCAP - 16)
        off = lax.fori_loop(0, _NVPT, comp, jnp.int32(0))

        cnt = jnp.minimum(off, _PCAP)
        cntv[...] = jnp.where(iota == 0, cnt, 0)
        pltpu.sync_copy(cands_s.at[pl.ds(0, _PCAP)], sh_cs.at[sid])
        pltpu.sync_copy(cands_i.at[pl.ds(0, _PCAP)], sh_ci.at[sid])
        pltpu.sync_copy(cntv, sh_cnt.at[sid, pl.ds(0, 16)])
        plsc.subcore_barrier()

        @pl.when(sid == 0)
        def _():
            pltpu.sync_copy(sh_cs, mbuf_s)
            pltpu.sync_copy(sh_ci, mbuf_i)
            pltpu.sync_copy(sh_cnt, mcnt)

            def mrow(s, moff):
                cnt_s = mcnt[s, pl.ds(0, 16)][0]

                def mv(k, mo):
                    v = mbuf_s[s, pl.ds(k * 16, 16)]
                    ii = mbuf_i[s, pl.ds(k * 16, 16)]
                    valid = (k * 16 + iota) < cnt_s
                    plsc.store_compressed(merged_s.at[pl.ds(mo, 16)], v, mask=valid)
                    plsc.store_compressed(merged_i.at[pl.ds(mo, 16)], ii, mask=valid)
                    pc = plsc.all_reduce_population_count(valid)
                    return mo + pc[0]
                return lax.fori_loop(0, _PCAP // 16, mv, moff)
            total = lax.fori_loop(0, 16, mrow, jnp.int32(0))

            merged_s[pl.ds(total, 16)] = jnp.full((16,), -1.0, jnp.float32)
            nv = (total + 15) // 16

            def zsel(k, c):
                sel_n[pl.ds(k * 16, 16)] = zeros16i
                return c
            lax.fori_loop(0, _SELP // 16, zsel, 0)

            def round_(r, c):
                def p1(v, macc):
                    return jnp.maximum(macc, merged_s[pl.ds(v * 16, 16)])
                macc = lax.fori_loop(0, nv, p1,
                                     jnp.full((16,), -2.0, jnp.float32))
                m = jnp.max(macc)

                def p2(v, iacc):
                    sv = merged_s[pl.ds(v * 16, 16)]
                    iv = merged_i[pl.ds(v * 16, 16)]
                    return jnp.minimum(iacc, jnp.where(sv == m, iv, _BIG))
                iacc = lax.fori_loop(0, nv, p2, jnp.full((16,), _BIG, jnp.int32))
                w = jnp.min(iacc)

                def p3(v, cc):
                    sv = merged_s[pl.ds(v * 16, 16)]
                    iv = merged_i[pl.ds(v * 16, 16)]
                    merged_s[pl.ds(v * 16, 16)] = jnp.where(
                        (sv == m) & (iv == w), -1.0, sv)
                    return cc
                lax.fori_loop(0, nv, p3, 0)
                plsc.store_scatter(sel_n, [zeros16i + r], zeros16i + w,
                                   mask=(iota == 0))
                return c
            lax.fori_loop(0, _K, round_, 0)

            for k in range(_SELP // 16):
                nv16 = sel_n[pl.ds(k * 16, 16)]
                sel_r[pl.ds(k * 16, 16)] = (
                    b * _ROWS + jnp.minimum(
                        lax.shift_right_logical(nv16, 7), _ROWS - 1))
            pltpu.async_copy(r0_hbm.at[sel_r], g0, sem).wait()
            pltpu.async_copy(r1_hbm.at[sel_r], g1, sem).wait()
            pltpu.async_copy(cf_hbm.at[sel_r], g2, sem).wait()

            for k in range(_SELP // 16):
                p = k * 16 + iota
                nv16 = sel_n[pl.ds(k * 16, 16)]
                col = nv16 & 127
                r0v = plsc.load_gather(g0, [p, col])
                r1v = plsc.load_gather(g1, [p, col])
                cfv = plsc.load_gather(g2, [p, col])
                gx = (nv16 // _HW).astype(jnp.float32)
                gy = (nv16 % _HW).astype(jnp.float32)
                plsc.store_scatter(outbuf, [3 * p + 0], gx + r0v)
                plsc.store_scatter(outbuf, [3 * p + 1], gy + r1v)
                plsc.store_scatter(outbuf, [3 * p + 2], cfv)
            pltpu.sync_copy(outbuf, out_hbm.at[b])
        plsc.subcore_barrier()


def _select(r0p, r1p, cfp, scp):
    mesh = plsc.VectorSubcoreMesh(core_axis_name="c", subcore_axis_name="s")
    f32, i32 = jnp.float32, jnp.int32
    fn = functools.partial(
        pl.kernel,
        mesh=mesh,
        out_type=jax.ShapeDtypeStruct((_B, _OPAD), f32),
        scratch_types=[
            pltpu.VMEM((_TROWS, 128), f32),
            pltpu.VMEM((1, _HB), i32),
            pltpu.VMEM((16,), i32),
            pltpu.VMEM((_CCAP,), f32),
            pltpu.VMEM((_CCAP,), i32),
            pltpu.VMEM((16,), i32),
            pltpu.VMEM((16, _PCAP), f32),
            pltpu.VMEM((16, _PCAP), i32),
            pltpu.VMEM((16, 128), i32),
            pltpu.VMEM((_MCAP,), f32),
            pltpu.VMEM((_MCAP,), i32),
            pltpu.VMEM((_SELP,), i32),
            pltpu.VMEM((_SELP,), i32),
            pltpu.VMEM((_SELP, 128), f32),
            pltpu.VMEM((_SELP, 128), f32),
            pltpu.VMEM((_SELP, 128), f32),
            pltpu.VMEM((_OPAD,), f32),
            pltpu.VMEM_SHARED((1, _HB), i32),
            pltpu.VMEM_SHARED((16, _PCAP), f32),
            pltpu.VMEM_SHARED((16, _PCAP), i32),
            pltpu.VMEM_SHARED((16, 128), i32),
            pltpu.SemaphoreType.DMA,
        ],
        compiler_params=pltpu.CompilerParams(needs_layout_passes=False),
    )(_sel_body)
    return fn(r0p, r1p, cfp, scp)


def kernel(x, W_reg, b_reg, W_conf, b_conf):
    B, C, H, W = x.shape
    r0p, r1p, cfp, scp = _compute_maps(x, W_reg, b_reg, W_conf, b_conf)
    padded = _select(r0p, r1p, cfp, scp)
    return padded[:, :3 * _K].reshape(B, _K, 3)


# speedup vs baseline: 2.8534x; 2.2552x over previous
import functools

import jax
import jax.numpy as jnp
from jax import lax
from jax.experimental import pallas as pl
from jax.experimental.pallas import tpu as pltpu
from jax.experimental.pallas import tpu_sc as plsc

_K = 100
_B = 4
_HW = 384
_N = _HW * _HW
_ROWS = _N // 128
_TROWS = _ROWS // 16
_NVPT = _TROWS * 8
_HB = 8192
_SHIFT = 19
_CCAP = 512
_PCAP = 128
_MCAP = 2080
_SELP = 112
_OPAD = 512
_BIG = 2**30


def _maps_kernel(w_ref, b_ref, x_ref, r0_ref, r1_ref, cf_ref, sc_ref):
    bh = x_ref.shape[2]
    bw = x_ref.shape[3]
    xb = x_ref[0].reshape(x_ref.shape[1], bh * bw)
    y = jnp.dot(w_ref[...], xb, preferred_element_type=jnp.float32)
    y = y + b_ref[...]
    reg0 = y[0]
    reg1 = y[1]
    conf = jax.nn.sigmoid(y[2])
    score = jnp.sqrt(reg0 * reg0 + reg1 * reg1) * conf
    r0_ref[0] = reg0.reshape(bh, bw)
    r1_ref[0] = reg1.reshape(bh, bw)
    cf_ref[0] = conf.reshape(bh, bw)
    sc_ref[0] = score.reshape(bh, bw)


def _compute_maps(x, W_reg, b_reg, W_conf, b_conf, bh=96, bw=128):
    B, C, H, W = x.shape
    w = jnp.zeros((8, C), jnp.float32).at[0:2].set(W_reg).at[2].set(W_conf[0])
    b = jnp.zeros((8, 1), jnp.float32).at[0:2, 0].set(b_reg).at[2, 0].set(b_conf[0])
    plane = jax.ShapeDtypeStruct((B, H, W), jnp.float32)
    out_spec = pl.BlockSpec((1, bh, bw), lambda b_, i, j: (b_, i, j))
    planes = pl.pallas_call(
        _maps_kernel,
        grid=(B, H // bh, W // bw),
        in_specs=[
            pl.BlockSpec((8, C), lambda b_, i, j: (0, 0)),
            pl.BlockSpec((8, 1), lambda b_, i, j: (0, 0)),
            pl.BlockSpec((1, C, bh, bw), lambda b_, i, j: (b_, 0, i, j)),
        ],
        out_specs=[out_spec, out_spec, out_spec, out_spec],
        out_shape=[plane, plane, plane, plane],
        compiler_params=pltpu.CompilerParams(
            dimension_semantics=("parallel", "arbitrary", "arbitrary"),
        ),
    )(w, b, x)
    return tuple(p.reshape(B * H * W // 128, 128) for p in planes)


def _sel_body(r0_hbm, r1_hbm, cf_hbm, sc_hbm, out_hbm,
              chunk, hist, zidx, cands_s, cands_i, cntv,
              mbuf_s, mbuf_i, mcnt, merged_s, merged_i,
              sel_n, sel_r, g0, g1, g2, outbuf,
              sh_hist, sh_cs, sh_ci, sh_cnt, sem):
    cid = lax.axis_index("c")
    sid = lax.axis_index("s")
    iota = lax.iota(jnp.int32, 16)
    zeros16i = jnp.zeros((16,), jnp.int32)
    ones16i = jnp.ones((16,), jnp.int32)

    zidx[...] = zeros16i

    for t in range(2):
        b = cid * 2 + t

        def zh(i, c):
            hist[0, pl.ds(i * 16, 16)] = zeros16i
            return c
        lax.fori_loop(0, _HB // 16, zh, 0)

        @pl.when(sid == 0)
        def _():
            pltpu.sync_copy(hist, sh_hist)

        row0 = b * _ROWS + sid * _TROWS
        pltpu.sync_copy(sc_hbm.at[pl.ds(row0, _TROWS), :], chunk)
        plsc.subcore_barrier()

        def hacc(i, c):
            r = i // 8
            c4 = (i % 8) * 16
            v = chunk[r, pl.ds(c4, 16)]
            bits = plsc.bitcast(v, jnp.int32)
            bkt = lax.shift_right_logical(bits, _SHIFT)
            plsc.addupdate_scatter(hist, [zeros16i, bkt], ones16i)
            return c
        lax.fori_loop(0, _NVPT, hacc, 0)

        pltpu.sync_copy(hist, sh_hist.at[zidx.at[pl.ds(0, 1)]], add=True)
        plsc.subcore_barrier()

        pltpu.sync_copy(sh_hist, hist)

        def scan_body(i, carry):
            acc, bstar = carry
            j = _HB // 16 - 1 - i
            v = hist[0, pl.ds(j * 16, 16)]
            rv = lax.rev(v, (0,))
            cs = jnp.cumsum(rv)
            suf = lax.rev(cs, (0,)) + acc
            bidx = j * 16 + iota
            cand = jnp.where(suf >= _K, bidx, -1)
            bstar = jnp.maximum(bstar, jnp.max(cand))
            return (acc + cs[15], bstar)

        _, bstar = lax.fori_loop(0, _HB // 16, scan_body,
                                 (jnp.int32(0), jnp.int32(-1)))
        thresh_bits = bstar << _SHIFT

        base_n = sid * (_N // 16)

        def comp(i, off):
            r = i // 8
            c4 = (i % 8) * 16
            v = chunk[r, pl.ds(c4, 16)]
            bits = plsc.bitcast(v, jnp.int32)
            msk = bits >= thresh_bits
            plsc.store_compressed(cands_s.at[pl.ds(off, 16)], v, mask=msk)
            gidx = base_n + i * 16 + iota
            plsc.store_compressed(cands_i.at[pl.ds(off, 16)], gidx, mask=msk)
            pc = plsc.all_reduce_population_count(msk)
            return jnp.minimum(off + pc[0], _CCAP - 16)
        off = lax.fori_loop(0, _NVPT, comp, jnp.int32(0))

        cnt = jnp.minimum(off, _PCAP)
        cntv[...] = jnp.where(iota == 0, cnt, 0)
        pltpu.sync_copy(cands_s.at[pl.ds(0, _PCAP)], sh_cs.at[sid])
        pltpu.sync_copy(cands_i.at[pl.ds(0, _PCAP)], sh_ci.at[sid])
        pltpu.sync_copy(cntv, sh_cnt.at[sid, pl.ds(0, 16)])
        plsc.subcore_barrier()

        @pl.when(sid == 0)
        def _():
            pltpu.sync_copy(sh_cs, mbuf_s)
            pltpu.sync_copy(sh_ci, mbuf_i)
            pltpu.sync_copy(sh_cnt, mcnt)

            def mrow(s, moff):
                cnt_s = mcnt[s, pl.ds(0, 16)][0]

                def mv(k, mo):
                    v = mbuf_s[s, pl.ds(k * 16, 16)]
                    ii = mbuf_i[s, pl.ds(k * 16, 16)]
                    valid = (k * 16 + iota) < cnt_s
                    plsc.store_compressed(merged_s.at[pl.ds(mo, 16)], v, mask=valid)
                    plsc.store_compressed(merged_i.at[pl.ds(mo, 16)], ii, mask=valid)
                    pc = plsc.all_reduce_population_count(valid)
                    return mo + pc[0]
                return lax.fori_loop(0, _PCAP // 16, mv, moff)
            total = lax.fori_loop(0, 16, mrow, jnp.int32(0))

            merged_s[pl.ds(total, 16)] = jnp.full((16,), -1.0, jnp.float32)
            nv = (total + 15) // 16

            def zsel(k, c):
                sel_n[pl.ds(k * 16, 16)] = zeros16i
                return c
            lax.fori_loop(0, _SELP // 16, zsel, 0)

            def round_(r, c):
                def p1(v, macc):
                    return jnp.maximum(macc, merged_s[pl.ds(v * 16, 16)])
                macc = lax.fori_loop(0, nv, p1,
                                     jnp.full((16,), -2.0, jnp.float32))
                m = jnp.max(macc)

                def p2(v, iacc):
                    sv = merged_s[pl.ds(v * 16, 16)]
                    iv = merged_i[pl.ds(v * 16, 16)]
                    return jnp.minimum(iacc, jnp.where(sv == m, iv, _BIG))
                iacc = lax.fori_loop(0, nv, p2, jnp.full((16,), _BIG, jnp.int32))
                w = jnp.min(iacc)

                def p3(v, cc):
                    sv = merged_s[pl.ds(v * 16, 16)]
                    iv = merged_i[pl.ds(v * 16, 16)]
                    merged_s[pl.ds(v * 16, 16)] = jnp.where(
                        (sv == m) & (iv == w), -1.0, sv)
                    return cc
                lax.fori_loop(0, nv, p3, 0)
                plsc.store_scatter(sel_n, [zeros16i + r], zeros16i + w,
                                   mask=(iota == 0))
                return c
            lax.fori_loop(0, _K, round_, 0)

            for k in range(_SELP // 16):
                nv16 = sel_n[pl.ds(k * 16, 16)]
                sel_r[pl.ds(k * 16, 16)] = (
                    b * _ROWS + jnp.minimum(
                        lax.shift_right_logical(nv16, 7), _ROWS - 1))
            pltpu.async_copy(r0_hbm.at[sel_r], g0, sem).wait()
            pltpu.async_copy(r1_hbm.at[sel_r], g1, sem).wait()
            pltpu.async_copy(cf_hbm.at[sel_r], g2, sem).wait()

            for k in range(_SELP // 16):
                p = k * 16 + iota
                nv16 = sel_n[pl.ds(k * 16, 16)]
                col = nv16 & 127
                r0v = plsc.load_gather(g0, [p, col])
                r1v = plsc.load_gather(g1, [p, col])
                cfv = plsc.load_gather(g2, [p, col])
                gx = (nv16 // _HW).astype(jnp.float32)
                gy = (nv16 % _HW).astype(jnp.float32)
                plsc.store_scatter(outbuf, [3 * p + 0], gx + r0v)
                plsc.store_scatter(outbuf, [3 * p + 1], gy + r1v)
                plsc.store_scatter(outbuf, [3 * p + 2], cfv)
            pltpu.sync_copy(outbuf, out_hbm.at[b])
        plsc.subcore_barrier()


def _select(r0p, r1p, cfp, scp):
    mesh = plsc.VectorSubcoreMesh(core_axis_name="c", subcore_axis_name="s")
    f32, i32 = jnp.float32, jnp.int32
    fn = functools.partial(
        pl.kernel,
        mesh=mesh,
        out_type=jax.ShapeDtypeStruct((_B, _OPAD), f32),
        scratch_types=[
            pltpu.VMEM((_TROWS, 128), f32),
            pltpu.VMEM((1, _HB), i32),
            pltpu.VMEM((16,), i32),
            pltpu.VMEM((_CCAP,), f32),
            pltpu.VMEM((_CCAP,), i32),
            pltpu.VMEM((16,), i32),
            pltpu.VMEM((16, _PCAP), f32),
            pltpu.VMEM((16, _PCAP), i32),
            pltpu.VMEM((16, 128), i32),
            pltpu.VMEM((_MCAP,), f32),
            pltpu.VMEM((_MCAP,), i32),
            pltpu.VMEM((_SELP,), i32),
            pltpu.VMEM((_SELP,), i32),
            pltpu.VMEM((_SELP, 128), f32),
            pltpu.VMEM((_SELP, 128), f32),
            pltpu.VMEM((_SELP, 128), f32),
            pltpu.VMEM((_OPAD,), f32),
            pltpu.VMEM_SHARED((1, _HB), i32),
            pltpu.VMEM_SHARED((16, _PCAP), f32),
            pltpu.VMEM_SHARED((16, _PCAP), i32),
            pltpu.VMEM_SHARED((16, 128), i32),
            pltpu.SemaphoreType.DMA,
        ],
        compiler_params=pltpu.CompilerParams(needs_layout_passes=False),
    )(_sel_body)
    return fn(r0p, r1p, cfp, scp)


def kernel(x, W_reg, b_reg, W_conf, b_conf):
    B, C, H, W = x.shape
    r0p, r1p, cfp, scp = _compute_maps(x, W_reg, b_reg, W_conf, b_conf)
    padded = _select(r0p, r1p, cfp, scp)
    return padded[:, :3 * _K].reshape(B, _K, 3)
